# baseline (speedup 1.0000x reference)
import functools

import jax
import jax.numpy as jnp
from jax import lax
from jax.experimental import pallas as pl
from jax.experimental.pallas import tpu as pltpu
from jax.experimental.pallas import tpu_sc as plsc

N = 10000
D = 128
TW = 144
E = 320000
NC, NS = 2, 16
NW = NC * NS
EW = E // NW
K = 50
NCH = EW // K
RPT = N // NS
VL = 16
NB_B = 4
NB_C = 2
NB_G = 4

GK = 100
GCH = (1024 * 50) // (NW * GK)
ROWBLK = 1000

_MESH = dict(core_axis_name="c", subcore_axis_name="s")
_SC_PARAMS = pltpu.CompilerParams(
    needs_layout_passes=False, use_tc_tiling_on_sc=False)


def _worker_id():
  return lax.axis_index("c") * NS + lax.axis_index("s")


@functools.partial(
    pl.kernel,
    out_type=jax.ShapeDtypeStruct((NW, N), jnp.float32),
    mesh=plsc.VectorSubcoreMesh(**_MESH),
    compiler_params=_SC_PARAMS,
    scratch_types=[
        pltpu.VMEM((EW,), jnp.int32),
        pltpu.VMEM((N,), jnp.float32),
        pltpu.SemaphoreType.DMA,
    ],
)
def _deg_pass(dst_hbm, out_hbm, idx_v, deg_v, sem):
  w = _worker_id()
  cp = pltpu.async_copy(dst_hbm.at[pl.ds(w * EW, EW)], idx_v, sem)

  def zero(i, carry):
    deg_v[pl.ds(i * VL, VL)] = jnp.zeros((VL,), jnp.float32)
    return carry

  lax.fori_loop(0, N // VL, zero, 0)
  cp.wait()

  ones = jnp.ones((VL,), jnp.float32)

  def scat(i, carry):
    idx = idx_v[pl.ds(i * VL, VL)]
    plsc.addupdate_scatter(deg_v, [idx], ones)
    return carry

  lax.fori_loop(0, EW // VL, scat, 0)
  pltpu.sync_copy(deg_v, out_hbm.at[w])


def _make_segsum(width, nb):
  @functools.partial(
      pl.kernel,
      out_type=jax.ShapeDtypeStruct((NC, N, width), jnp.float32),
      mesh=plsc.VectorSubcoreMesh(**_MESH),
      compiler_params=_SC_PARAMS,
      scratch_types=(
          [
              pltpu.VMEM((NCH, K), jnp.int32),
              pltpu.VMEM((NCH, K), jnp.int32),
              pltpu.VMEM_SHARED((N, width), jnp.float32),
          ]
          + [pltpu.VMEM((K, width), jnp.float32) for _ in range(nb)]
          + [pltpu.SemaphoreType.DMA for _ in range(2 * nb + 2)]
      ),
  )
  def seg(tab_hbm, src2_hbm, dst2_hbm, out_hbm, sidx, didx, acc, *rs):
    rows = rs[:nb]
    gsem = rs[nb:2 * nb]
    ssem = rs[2 * nb:3 * nb]
    isem = rs[3 * nb]
    zsem = rs[3 * nb + 1]
    c = lax.axis_index("c")
    s = lax.axis_index("s")
    w = c * NS + s
    cp_s = pltpu.async_copy(src2_hbm.at[pl.ds(w * NCH, NCH)], sidx, isem)
    cp_d = pltpu.async_copy(dst2_hbm.at[pl.ds(w * NCH, NCH)], didx, isem)

    def zrow(i, carry):
      for ch in range(width // VL):
        rows[0][i, pl.ds(ch * VL, VL)] = jnp.zeros((VL,), jnp.float32)
      return carry

    lax.fori_loop(0, K, zrow, 0)
    row0 = s * RPT
    zc = []
    for r in range(RPT // K):
      zc.append(pltpu.async_copy(
          rows[0], acc.at[pl.ds(row0 + r * K, K)], zsem))
    if RPT % K:
      zc.append(pltpu.async_copy(
          rows[0].at[pl.ds(0, RPT % K)],
          acc.at[pl.ds(row0 + (RPT // K) * K, RPT % K)], zsem))
    cp_s.wait()
    for b in range(1, nb):
      pltpu.async_copy(tab_hbm.at[sidx.at[b]], rows[b], gsem[b])
    for z in zc:
      z.wait()
    pltpu.async_copy(tab_hbm.at[sidx.at[0]], rows[0], gsem[0])
    cp_d.wait()
    plsc.subcore_barrier()

    def body(jj, carry):
      base = jj * nb
      for b in range(nb):
        pltpu.make_async_copy(
            tab_hbm.at[sidx.at[base + b]], rows[b], gsem[b]).wait()
        pltpu.async_copy(rows[b], acc.at[didx.at[base + b]], ssem[b], add=True)
      for b in range(nb):
        nxt = base + nb + b

        @pl.when(nxt < NCH)
        def _():
          pltpu.make_async_copy(
              rows[b], acc.at[didx.at[base + b]], ssem[b]).wait()
          pltpu.async_copy(tab_hbm.at[sidx.at[nxt]], rows[b], gsem[b])

      return carry

    lax.fori_loop(0, NCH // nb, body, 0)
    for b in range(nb):
      pltpu.make_async_copy(
          rows[b], acc.at[didx.at[NCH - nb + b]], ssem[b]).wait()
    plsc.subcore_barrier()
    pltpu.sync_copy(acc.at[pl.ds(row0, RPT)], out_hbm.at[c, pl.ds(row0, RPT)])

  return seg


_segsum_b = _make_segsum(D, NB_B)
_segsum_c = _make_segsum(TW, NB_C)


@functools.partial(
    pl.kernel,
    out_type=jax.ShapeDtypeStruct((1024 * 50, D), jnp.float32),
    mesh=plsc.VectorSubcoreMesh(**_MESH),
    compiler_params=_SC_PARAMS,
    scratch_types=(
        [pltpu.VMEM((GCH, GK), jnp.int32)]
        + [pltpu.VMEM((GK, D), jnp.float32) for _ in range(NB_G)]
        + [pltpu.SemaphoreType.DMA for _ in range(2 * NB_G)]
    ),
)
def _gather_pass(g_hbm, idx2_hbm, out_hbm, idxv, *rs):
  rows = rs[:NB_G]
  gsem = rs[NB_G:2 * NB_G]
  osem = rs[2 * NB_G:3 * NB_G]
  w = _worker_id()
  pltpu.sync_copy(idx2_hbm.at[pl.ds(w * GCH, GCH)], idxv)
  for b in range(NB_G):
    pltpu.async_copy(g_hbm.at[idxv.at[b]], rows[b], gsem[b])

  def body(jj, carry):
    base = jj * NB_G
    for b in range(NB_G):
      pltpu.make_async_copy(g_hbm.at[idxv.at[base + b]], rows[b],
                            gsem[b]).wait()
      pltpu.async_copy(
          rows[b], out_hbm.at[pl.ds((w * GCH + base + b) * GK, GK)], osem[b])
    for b in range(NB_G):
      nxt = base + NB_G + b

      @pl.when(nxt < GCH)
      def _():
        pltpu.make_async_copy(
            rows[b], out_hbm.at[pl.ds((w * GCH + base + b) * GK, GK)],
            osem[b]).wait()
        pltpu.async_copy(g_hbm.at[idxv.at[nxt]], rows[b], gsem[b])

    return carry

  lax.fori_loop(0, GCH // NB_G, body, 0)
  for b in range(NB_G):
    pltpu.make_async_copy(
        rows[b], out_hbm.at[pl.ds((w * GCH + GCH - NB_G + b) * GK, GK)],
        osem[b]).wait()


def _tc1_body(parts_ref, emb_ref, u_ref, dinv_ref):
  ones = jnp.ones((NW, 1), jnp.float32)
  deg = lax.dot_general(
      parts_ref[...], ones, (((0,), (0,)), ((), ())),
      preferred_element_type=jnp.float32) + 1.0
  dinv = lax.rsqrt(deg)
  u_ref[...] = emb_ref[...] * dinv
  dinv_ref[...] = dinv


def _tc1(parts, emb):
  return pl.pallas_call(
      _tc1_body,
      out_shape=[
          jax.ShapeDtypeStruct((N, D), jnp.float32),
          jax.ShapeDtypeStruct((N, 1), jnp.float32),
      ],
  )(parts, emb)


def _tc2_body(acc_ref, u_ref, dinv_ref, w1_ref, b1_ref, ttab_ref, r_ref):
  s = acc_ref[0] + acc_ref[1]
  dinv = dinv_ref[...]
  su = s + u_ref[...]
  h = dinv * jnp.dot(su, w1_ref[...],
                     preferred_element_type=jnp.float32) + b1_ref[...]
  pad = jnp.zeros((su.shape[0], TW - D - 1), jnp.float32)
  ttab_ref[...] = jnp.concatenate([(dinv * dinv) * su, dinv, pad], axis=1)
  r_ref[...] = dinv * h


def _tc2(accB, u, dinvcol, W1, b1row):
  nb = N // ROWBLK
  return pl.pallas_call(
      _tc2_body,
      grid=(nb,),
      in_specs=[
          pl.BlockSpec((NC, ROWBLK, D), lambda i: (0, i, 0)),
          pl.BlockSpec((ROWBLK, D), lambda i: (i, 0)),
          pl.BlockSpec((ROWBLK, 1), lambda i: (i, 0)),
          pl.BlockSpec((D, 2 * D), lambda i: (0, 0)),
          pl.BlockSpec((1, 2 * D), lambda i: (0, 0)),
      ],
      out_specs=[
          pl.BlockSpec((ROWBLK, TW), lambda i: (i, 0)),
          pl.BlockSpec((ROWBLK, 2 * D), lambda i: (i, 0)),
      ],
      out_shape=[
          jax.ShapeDtypeStruct((N, TW), jnp.float32),
          jax.ShapeDtypeStruct((N, 2 * D), jnp.float32),
      ],
  )(accB, u, dinvcol, W1, b1row)


def _tc3_body(acc_ref, r_ref, dinv_ref, w1_ref, b1_ref, w2_ref, b2_ref, g_ref):
  a = acc_ref[0] + acc_ref[1]
  s2a = a[:, :D]
  cc = a[:, D:D + 1]
  z = jnp.dot(s2a, w1_ref[...], preferred_element_type=jnp.float32)
  z = z + cc * b1_ref[...] + r_ref[...]
  g_ref[...] = dinv_ref[...] * jnp.dot(
      z, w2_ref[...], preferred_element_type=jnp.float32) + b2_ref[...]


def _tc3(accC, r, dinvcol, W1, b1row, W2, b2row):
  nb = N // ROWBLK
  return pl.pallas_call(
      _tc3_body,
      grid=(nb,),
      in_specs=[
          pl.BlockSpec((NC, ROWBLK, TW), lambda i: (0, i, 0)),
          pl.BlockSpec((ROWBLK, 2 * D), lambda i: (i, 0)),
          pl.BlockSpec((ROWBLK, 1), lambda i: (i, 0)),
          pl.BlockSpec((D, 2 * D), lambda i: (0, 0)),
          pl.BlockSpec((1, 2 * D), lambda i: (0, 0)),
          pl.BlockSpec((2 * D, D), lambda i: (0, 0)),
          pl.BlockSpec((1, D), lambda i: (0, 0)),
      ],
      out_specs=pl.BlockSpec((ROWBLK, D), lambda i: (i, 0)),
      out_shape=jax.ShapeDtypeStruct((N, D), jnp.float32),
  )(accC, r, dinvcol, W1, b1row, W2, b2row)


def kernel(input, input_timestamp, input_id, edge_index, emb, W1, b1, W2, b2):
  del input_timestamp, input_id
  src2 = edge_index[0].astype(jnp.int32).reshape(E // K, K)
  dst = edge_index[1].astype(jnp.int32)
  dst2 = dst.reshape(E // K, K)

  deg_parts = _deg_pass(dst)
  u, dinvcol = _tc1(deg_parts, emb)
  accB = _segsum_b(u, src2, dst2)
  ttab, r = _tc2(accB, u, dinvcol, W1, b1.reshape(1, -1))
  accC = _segsum_c(ttab, src2, dst2)
  g = _tc3(accC, r, dinvcol, W1, b1.reshape(1, -1), W2, b2.reshape(1, -1))

  bsz, mlen = input.shape
  idx2 = input.reshape(-1).astype(jnp.int32).reshape(-1, GK)
  out = _gather_pass(g, idx2)
  return out.reshape(bsz, mlen, D)


# speedup vs baseline: 1.0893x; 1.0893x over previous
import functools

import jax
import jax.numpy as jnp
from jax import lax
from jax.experimental import pallas as pl
from jax.experimental.pallas import tpu as pltpu
from jax.experimental.pallas import tpu_sc as plsc

N = 10000
D = 128
E = 320000
NC, NS = 2, 16
NW = NC * NS
EW = E // NW
K = 100
NCH = EW // K
RPT = N // NS
VL = 16
NBR = 3
NFULL = (NCH // NBR) * NBR
REM = NCH - NFULL
NB_G = 4

GK = 100
GCH = (1024 * 50) // (NW * GK)
ROWBLK = 1000

_MESH = dict(core_axis_name="c", subcore_axis_name="s")
_SC_PARAMS = pltpu.CompilerParams(
    needs_layout_passes=False, use_tc_tiling_on_sc=False)


def _worker_id():
  return lax.axis_index("c") * NS + lax.axis_index("s")


@functools.partial(
    pl.kernel,
    out_type=jax.ShapeDtypeStruct((NW, N), jnp.float32),
    mesh=plsc.VectorSubcoreMesh(**_MESH),
    compiler_params=_SC_PARAMS,
    scratch_types=[
        pltpu.VMEM((EW,), jnp.int32),
        pltpu.VMEM((N,), jnp.float32),
        pltpu.SemaphoreType.DMA,
    ],
)
def _deg_pass(dst_hbm, out_hbm, idx_v, deg_v, sem):
  w = _worker_id()
  cp = pltpu.async_copy(dst_hbm.at[pl.ds(w * EW, EW)], idx_v, sem)

  def zero(i, carry):
    deg_v[pl.ds(i * VL, VL)] = jnp.zeros((VL,), jnp.float32)
    return carry

  lax.fori_loop(0, N // VL, zero, 0)
  cp.wait()

  ones = jnp.ones((VL,), jnp.float32)

  def scat(i, carry):
    idx = idx_v[pl.ds(i * VL, VL)]
    plsc.addupdate_scatter(deg_v, [idx], ones)
    return carry

  lax.fori_loop(0, EW // VL, scat, 0)
  pltpu.sync_copy(deg_v, out_hbm.at[w])


@functools.partial(
    pl.kernel,
    out_type=jax.ShapeDtypeStruct((NW, N), jnp.float32),
    mesh=plsc.VectorSubcoreMesh(**_MESH),
    compiler_params=_SC_PARAMS,
    scratch_types=[
        pltpu.VMEM((EW,), jnp.int32),
        pltpu.VMEM((EW,), jnp.int32),
        pltpu.VMEM((N,), jnp.float32),
        pltpu.VMEM((N,), jnp.float32),
        pltpu.SemaphoreType.DMA,
    ],
)
def _c_pass(dinv_hbm, src_hbm, dst_hbm, out_hbm, sidx, didx, dv, cacc, sem):
  w = _worker_id()
  cp0 = pltpu.async_copy(dinv_hbm, dv, sem)
  cp1 = pltpu.async_copy(src_hbm.at[pl.ds(w * EW, EW)], sidx, sem)
  cp2 = pltpu.async_copy(dst_hbm.at[pl.ds(w * EW, EW)], didx, sem)

  def zero(i, carry):
    cacc[pl.ds(i * VL, VL)] = jnp.zeros((VL,), jnp.float32)
    return carry

  lax.fori_loop(0, N // VL, zero, 0)
  cp0.wait()
  cp1.wait()
  cp2.wait()

  def scat(i, carry):
    si = sidx[pl.ds(i * VL, VL)]
    vals = plsc.load_gather(dv, [si])
    di = didx[pl.ds(i * VL, VL)]
    plsc.addupdate_scatter(cacc, [di], vals)
    return carry

  lax.fori_loop(0, EW // VL, scat, 0)
  pltpu.sync_copy(cacc, out_hbm.at[w])


@functools.partial(
    pl.kernel,
    out_type=jax.ShapeDtypeStruct((NC, N, D), jnp.float32),
    mesh=plsc.VectorSubcoreMesh(**_MESH),
    compiler_params=_SC_PARAMS,
    scratch_types=(
        [pltpu.VMEM_SHARED((N, D), jnp.float32)]
        + [pltpu.VMEM((K,), jnp.int32) for _ in range(2 * NBR)]
        + [pltpu.VMEM((K, D), jnp.float32) for _ in range(NBR)]
        + [pltpu.SemaphoreType.DMA for _ in range(3 * NBR + 1)]
    ),
)
def _segsum(tab_hbm, src2_hbm, dst2_hbm, out_hbm, acc, *rs):
  sbuf = rs[:NBR]
  dbuf = rs[NBR:2 * NBR]
  rows = rs[2 * NBR:3 * NBR]
  xsem = rs[3 * NBR:4 * NBR]
  gsem = rs[4 * NBR:5 * NBR]
  ssem = rs[5 * NBR:6 * NBR]
  zsem = rs[6 * NBR]
  c = lax.axis_index("c")
  s = lax.axis_index("s")
  w = c * NS + s
  ch0 = w * NCH

  def idx_start(j, b):
    pltpu.async_copy(src2_hbm.at[ch0 + j], sbuf[b], xsem[b])
    pltpu.async_copy(dst2_hbm.at[ch0 + j], dbuf[b], xsem[b])

  def idx_wait(j, b):
    pltpu.make_async_copy(src2_hbm.at[ch0 + j], sbuf[b], xsem[b]).wait()
    pltpu.make_async_copy(dst2_hbm.at[ch0 + j], dbuf[b], xsem[b]).wait()

  def gather_start(b):
    pltpu.async_copy(tab_hbm.at[sbuf[b]], rows[b], gsem[b])

  def gather_wait(b):
    pltpu.make_async_copy(tab_hbm.at[sbuf[b]], rows[b], gsem[b]).wait()

  def scat_start(b):
    pltpu.async_copy(rows[b], acc.at[dbuf[b]], ssem[b], add=True)

  def scat_wait(b):
    pltpu.make_async_copy(rows[b], acc.at[dbuf[b]], ssem[b]).wait()

  for b in range(NBR):
    idx_start(b, b)

  def zrow(i, carry):
    for ch in range(D // VL):
      rows[0][i, pl.ds(ch * VL, VL)] = jnp.zeros((VL,), jnp.float32)
    return carry

  lax.fori_loop(0, K, zrow, 0)
  row0 = s * RPT
  zc = []
  for r in range(RPT // K):
    zc.append(pltpu.async_copy(
        rows[0], acc.at[pl.ds(row0 + r * K, K)], zsem))
  if RPT % K:
    zc.append(pltpu.async_copy(
        rows[0].at[pl.ds(0, RPT % K)],
        acc.at[pl.ds(row0 + (RPT // K) * K, RPT % K)], zsem))
  for b in range(1, NBR):
    idx_wait(b, b)
    gather_start(b)
  for z in zc:
    z.wait()
  idx_wait(0, 0)
  gather_start(0)
  plsc.subcore_barrier()

  def body(jj, carry):
    base = jj * NBR
    for b in range(NBR):
      gather_wait(b)
      scat_start(b)
    for b in range(NBR):
      nxt = base + NBR + b

      @pl.when(nxt < NCH)
      def _():
        scat_wait(b)
        idx_start(nxt, b)

    for b in range(NBR):
      nxt = base + NBR + b

      @pl.when(nxt < NCH)
      def _():
        idx_wait(nxt, b)
        gather_start(b)

    return carry

  lax.fori_loop(0, NCH // NBR, body, 0)
  for t in range(REM):
    gather_wait(t)
    scat_start(t)
  for b in range(NBR):
    scat_wait(b)
  plsc.subcore_barrier()
  pltpu.sync_copy(acc.at[pl.ds(row0, RPT)], out_hbm.at[c, pl.ds(row0, RPT)])


@functools.partial(
    pl.kernel,
    out_type=jax.ShapeDtypeStruct((1024 * 50, D), jnp.float32),
    mesh=plsc.VectorSubcoreMesh(**_MESH),
    compiler_params=_SC_PARAMS,
    scratch_types=(
        [pltpu.VMEM((GCH, GK), jnp.int32)]
        + [pltpu.VMEM((GK, D), jnp.float32) for _ in range(NB_G)]
        + [pltpu.SemaphoreType.DMA for _ in range(2 * NB_G)]
    ),
)
def _gather_pass(g_hbm, idx2_hbm, out_hbm, idxv, *rs):
  rows = rs[:NB_G]
  gsem = rs[NB_G:2 * NB_G]
  osem = rs[2 * NB_G:3 * NB_G]
  w = _worker_id()
  pltpu.sync_copy(idx2_hbm.at[pl.ds(w * GCH, GCH)], idxv)
  for b in range(NB_G):
    pltpu.async_copy(g_hbm.at[idxv.at[b]], rows[b], gsem[b])

  def body(jj, carry):
    base = jj * NB_G
    for b in range(NB_G):
      pltpu.make_async_copy(g_hbm.at[idxv.at[base + b]], rows[b],
                            gsem[b]).wait()
      pltpu.async_copy(
          rows[b], out_hbm.at[pl.ds((w * GCH + base + b) * GK, GK)], osem[b])
    for b in range(NB_G):
      nxt = base + NB_G + b

      @pl.when(nxt < GCH)
      def _():
        pltpu.make_async_copy(
            rows[b], out_hbm.at[pl.ds((w * GCH + base + b) * GK, GK)],
            osem[b]).wait()
        pltpu.async_copy(g_hbm.at[idxv.at[nxt]], rows[b], gsem[b])

    return carry

  lax.fori_loop(0, GCH // NB_G, body, 0)
  for b in range(NB_G):
    pltpu.make_async_copy(
        rows[b], out_hbm.at[pl.ds((w * GCH + GCH - NB_G + b) * GK, GK)],
        osem[b]).wait()


def _tc1_body(parts_ref, emb_ref, u_ref, dinv_ref):
  ones = jnp.ones((NW, 1), jnp.float32)
  deg = lax.dot_general(
      parts_ref[...], ones, (((0,), (0,)), ((), ())),
      preferred_element_type=jnp.float32) + 1.0
  dinv = lax.rsqrt(deg)
  u_ref[...] = emb_ref[...] * dinv
  dinv_ref[...] = dinv


def _tc1(parts, emb):
  return pl.pallas_call(
      _tc1_body,
      out_shape=[
          jax.ShapeDtypeStruct((N, D), jnp.float32),
          jax.ShapeDtypeStruct((N, 1), jnp.float32),
      ],
  )(parts, emb)


def _tcc_body(parts_ref, c_ref):
  ones = jnp.ones((NW, 1), jnp.float32)
  c_ref[...] = lax.dot_general(
      parts_ref[...], ones, (((0,), (0,)), ((), ())),
      preferred_element_type=jnp.float32)


def _tcc(parts):
  return pl.pallas_call(
      _tcc_body,
      out_shape=jax.ShapeDtypeStruct((N, 1), jnp.float32),
  )(parts)


def _tc2_body(acc_ref, u_ref, dinv_ref, c_ref, w1_ref, b1_ref, ttab_ref, r_ref):
  s = acc_ref[0] + acc_ref[1]
  dinv = dinv_ref[...]
  su = s + u_ref[...]
  b1 = b1_ref[...]
  h = dinv * jnp.dot(su, w1_ref[...], preferred_element_type=jnp.float32) + b1
  ttab_ref[...] = (dinv * dinv) * su
  r_ref[...] = dinv * h + c_ref[...] * b1


def _tc2(accB, u, dinvcol, ccol, W1, b1row):
  nb = N // ROWBLK
  return pl.pallas_call(
      _tc2_body,
      grid=(nb,),
      in_specs=[
          pl.BlockSpec((NC, ROWBLK, D), lambda i: (0, i, 0)),
          pl.BlockSpec((ROWBLK, D), lambda i: (i, 0)),
          pl.BlockSpec((ROWBLK, 1), lambda i: (i, 0)),
          pl.BlockSpec((ROWBLK, 1), lambda i: (i, 0)),
          pl.BlockSpec((D, 2 * D), lambda i: (0, 0)),
          pl.BlockSpec((1, 2 * D), lambda i: (0, 0)),
      ],
      out_specs=[
          pl.BlockSpec((ROWBLK, D), lambda i: (i, 0)),
          pl.BlockSpec((ROWBLK, 2 * D), lambda i: (i, 0)),
      ],
      out_shape=[
          jax.ShapeDtypeStruct((N, D), jnp.float32),
          jax.ShapeDtypeStruct((N, 2 * D), jnp.float32),
      ],
  )(accB, u, dinvcol, ccol, W1, b1row)


def _tc3_body(acc_ref, r_ref, dinv_ref, w1_ref, w2_ref, b2_ref, g_ref):
  s2a = acc_ref[0] + acc_ref[1]
  z = jnp.dot(s2a, w1_ref[...], preferred_element_type=jnp.float32) + r_ref[...]
  g_ref[...] = dinv_ref[...] * jnp.dot(
      z, w2_ref[...], preferred_element_type=jnp.float32) + b2_ref[...]


def _tc3(accC, r, dinvcol, W1, W2, b2row):
  nb = N // ROWBLK
  return pl.pallas_call(
      _tc3_body,
      grid=(nb,),
      in_specs=[
          pl.BlockSpec((NC, ROWBLK, D), lambda i: (0, i, 0)),
          pl.BlockSpec((ROWBLK, 2 * D), lambda i: (i, 0)),
          pl.BlockSpec((ROWBLK, 1), lambda i: (i, 0)),
          pl.BlockSpec((D, 2 * D), lambda i: (0, 0)),
          pl.BlockSpec((2 * D, D), lambda i: (0, 0)),
          pl.BlockSpec((1, D), lambda i: (0, 0)),
      ],
      out_specs=pl.BlockSpec((ROWBLK, D), lambda i: (i, 0)),
      out_shape=jax.ShapeDtypeStruct((N, D), jnp.float32),
  )(accC, r, dinvcol, W1, W2, b2row)


def kernel(input, input_timestamp, input_id, edge_index, emb, W1, b1, W2, b2):
  del input_timestamp, input_id
  src = edge_index[0].astype(jnp.int32)
  dst = edge_index[1].astype(jnp.int32)
  src2 = src.reshape(E // K, K)
  dst2 = dst.reshape(E // K, K)

  deg_parts = _deg_pass(dst)
  u, dinvcol = _tc1(deg_parts, emb)
  accB = _segsum(u, src2, dst2)
  c_parts = _c_pass(dinvcol.reshape(N), src, dst)
  ccol = _tcc(c_parts)
  ttab, r = _tc2(accB, u, dinvcol, ccol, W1, b1.reshape(1, -1))
  accC = _segsum(ttab, src2, dst2)
  g = _tc3(accC, r, dinvcol, W1, W2, b2.reshape(1, -1))

  bsz, mlen = input.shape
  idx2 = input.reshape(-1).astype(jnp.int32).reshape(-1, GK)
  out = _gather_pass(g, idx2)
  return out.reshape(bsz, mlen, D)


# speedup vs baseline: 1.1101x; 1.0191x over previous
import functools

import jax
import jax.numpy as jnp
from jax import lax
from jax.experimental import pallas as pl
from jax.experimental.pallas import tpu as pltpu
from jax.experimental.pallas import tpu_sc as plsc

N = 10000
D = 128
E = 320000
NC, NS = 2, 16
NW = NC * NS
EW = E // NW
K = 50
NCH = EW // K
RPT = N // NS
VL = 16
NBR = 6
NFULL = (NCH // NBR) * NBR
REM = NCH - NFULL
NB_G = 4

GK = 100
GCH = (1024 * 50) // (NW * GK)
ROWBLK = 1000

_MESH = dict(core_axis_name="c", subcore_axis_name="s")
_SC_PARAMS = pltpu.CompilerParams(
    needs_layout_passes=False, use_tc_tiling_on_sc=False)


def _worker_id():
  return lax.axis_index("c") * NS + lax.axis_index("s")


@functools.partial(
    pl.kernel,
    out_type=jax.ShapeDtypeStruct((NW, N), jnp.float32),
    mesh=plsc.VectorSubcoreMesh(**_MESH),
    compiler_params=_SC_PARAMS,
    scratch_types=[
        pltpu.VMEM((EW,), jnp.int32),
        pltpu.VMEM((N,), jnp.float32),
        pltpu.SemaphoreType.DMA,
    ],
)
def _deg_pass(dst_hbm, out_hbm, idx_v, deg_v, sem):
  w = _worker_id()
  cp = pltpu.async_copy(dst_hbm.at[pl.ds(w * EW, EW)], idx_v, sem)

  def zero(i, carry):
    deg_v[pl.ds(i * VL, VL)] = jnp.zeros((VL,), jnp.float32)
    return carry

  lax.fori_loop(0, N // VL, zero, 0)
  cp.wait()

  ones = jnp.ones((VL,), jnp.float32)

  def scat(i, carry):
    idx = idx_v[pl.ds(i * VL, VL)]
    plsc.addupdate_scatter(deg_v, [idx], ones)
    return carry

  lax.fori_loop(0, EW // VL, scat, 0)
  pltpu.sync_copy(deg_v, out_hbm.at[w])


@functools.partial(
    pl.kernel,
    out_type=jax.ShapeDtypeStruct((NW, N), jnp.float32),
    mesh=plsc.VectorSubcoreMesh(**_MESH),
    compiler_params=_SC_PARAMS,
    scratch_types=[
        pltpu.VMEM((EW,), jnp.int32),
        pltpu.VMEM((EW,), jnp.int32),
        pltpu.VMEM((N,), jnp.float32),
        pltpu.VMEM((N,), jnp.float32),
        pltpu.SemaphoreType.DMA,
    ],
)
def _c_pass(dinv_hbm, src_hbm, dst_hbm, out_hbm, sidx, didx, dv, cacc, sem):
  w = _worker_id()
  cp0 = pltpu.async_copy(dinv_hbm, dv, sem)
  cp1 = pltpu.async_copy(src_hbm.at[pl.ds(w * EW, EW)], sidx, sem)
  cp2 = pltpu.async_copy(dst_hbm.at[pl.ds(w * EW, EW)], didx, sem)

  def zero(i, carry):
    cacc[pl.ds(i * VL, VL)] = jnp.zeros((VL,), jnp.float32)
    return carry

  lax.fori_loop(0, N // VL, zero, 0)
  cp0.wait()
  cp1.wait()
  cp2.wait()

  def scat(i, carry):
    si = sidx[pl.ds(i * VL, VL)]
    vals = plsc.load_gather(dv, [si])
    di = didx[pl.ds(i * VL, VL)]
    plsc.addupdate_scatter(cacc, [di], vals)
    return carry

  lax.fori_loop(0, EW // VL, scat, 0)
  pltpu.sync_copy(cacc, out_hbm.at[w])


@functools.partial(
    pl.kernel,
    out_type=jax.ShapeDtypeStruct((NC, N, D), jnp.float32),
    mesh=plsc.VectorSubcoreMesh(**_MESH),
    compiler_params=_SC_PARAMS,
    scratch_types=(
        [pltpu.VMEM_SHARED((N, D), jnp.float32)]
        + [pltpu.VMEM((K,), jnp.int32) for _ in range(2 * NBR)]
        + [pltpu.VMEM((K, D), jnp.float32) for _ in range(NBR)]
        + [pltpu.SemaphoreType.DMA for _ in range(3 * NBR + 1)]
    ),
)
def _segsum(tab_hbm, src2_hbm, dst2_hbm, out_hbm, acc, *rs):
  sbuf = rs[:NBR]
  dbuf = rs[NBR:2 * NBR]
  rows = rs[2 * NBR:3 * NBR]
  xsem = rs[3 * NBR:4 * NBR]
  gsem = rs[4 * NBR:5 * NBR]
  ssem = rs[5 * NBR:6 * NBR]
  zsem = rs[6 * NBR]
  c = lax.axis_index("c")
  s = lax.axis_index("s")
  w = c * NS + s
  ch0 = w * NCH

  def idx_start(j, b):
    pltpu.async_copy(src2_hbm.at[ch0 + j], sbuf[b], xsem[b])
    pltpu.async_copy(dst2_hbm.at[ch0 + j], dbuf[b], xsem[b])

  def idx_wait(j, b):
    pltpu.make_async_copy(src2_hbm.at[ch0 + j], sbuf[b], xsem[b]).wait()
    pltpu.make_async_copy(dst2_hbm.at[ch0 + j], dbuf[b], xsem[b]).wait()

  def gather_start(b):
    pltpu.async_copy(tab_hbm.at[sbuf[b]], rows[b], gsem[b])

  def gather_wait(b):
    pltpu.make_async_copy(tab_hbm.at[sbuf[b]], rows[b], gsem[b]).wait()

  def scat_start(b):
    pltpu.async_copy(rows[b], acc.at[dbuf[b]], ssem[b], add=True)

  def scat_wait(b):
    pltpu.make_async_copy(rows[b], acc.at[dbuf[b]], ssem[b]).wait()

  for b in range(NBR):
    idx_start(b, b)

  def zrow(i, carry):
    for ch in range(D // VL):
      rows[0][i, pl.ds(ch * VL, VL)] = jnp.zeros((VL,), jnp.float32)
    return carry

  lax.fori_loop(0, K, zrow, 0)
  row0 = s * RPT
  zc = []
  for r in range(RPT // K):
    zc.append(pltpu.async_copy(
        rows[0], acc.at[pl.ds(row0 + r * K, K)], zsem))
  if RPT % K:
    zc.append(pltpu.async_copy(
        rows[0].at[pl.ds(0, RPT % K)],
        acc.at[pl.ds(row0 + (RPT // K) * K, RPT % K)], zsem))
  for b in range(1, NBR):
    idx_wait(b, b)
    gather_start(b)
  for z in zc:
    z.wait()
  idx_wait(0, 0)
  gather_start(0)
  plsc.subcore_barrier()

  def body(jj, carry):
    base = jj * NBR
    for b in range(NBR):
      gather_wait(b)
      scat_start(b)
    for b in range(NBR):
      nxt = base + NBR + b

      @pl.when(nxt < NCH)
      def _():
        scat_wait(b)
        idx_start(nxt, b)

    for b in range(NBR):
      nxt = base + NBR + b

      @pl.when(nxt < NCH)
      def _():
        idx_wait(nxt, b)
        gather_start(b)

    return carry

  lax.fori_loop(0, NCH // NBR, body, 0)
  for t in range(REM):
    gather_wait(t)
    scat_start(t)
  for b in range(NBR):
    scat_wait(b)
  plsc.subcore_barrier()
  pltpu.sync_copy(acc.at[pl.ds(row0, RPT)], out_hbm.at[c, pl.ds(row0, RPT)])


@functools.partial(
    pl.kernel,
    out_type=jax.ShapeDtypeStruct((1024 * 50, D), jnp.float32),
    mesh=plsc.VectorSubcoreMesh(**_MESH),
    compiler_params=_SC_PARAMS,
    scratch_types=(
        [pltpu.VMEM((GCH, GK), jnp.int32)]
        + [pltpu.VMEM((GK, D), jnp.float32) for _ in range(NB_G)]
        + [pltpu.SemaphoreType.DMA for _ in range(2 * NB_G)]
    ),
)
def _gather_pass(g_hbm, idx2_hbm, out_hbm, idxv, *rs):
  rows = rs[:NB_G]
  gsem = rs[NB_G:2 * NB_G]
  osem = rs[2 * NB_G:3 * NB_G]
  w = _worker_id()
  pltpu.sync_copy(idx2_hbm.at[pl.ds(w * GCH, GCH)], idxv)
  for b in range(NB_G):
    pltpu.async_copy(g_hbm.at[idxv.at[b]], rows[b], gsem[b])

  def body(jj, carry):
    base = jj * NB_G
    for b in range(NB_G):
      pltpu.make_async_copy(g_hbm.at[idxv.at[base + b]], rows[b],
                            gsem[b]).wait()
      pltpu.async_copy(
          rows[b], out_hbm.at[pl.ds((w * GCH + base + b) * GK, GK)], osem[b])
    for b in range(NB_G):
      nxt = base + NB_G + b

      @pl.when(nxt < GCH)
      def _():
        pltpu.make_async_copy(
            rows[b], out_hbm.at[pl.ds((w * GCH + base + b) * GK, GK)],
            osem[b]).wait()
        pltpu.async_copy(g_hbm.at[idxv.at[nxt]], rows[b], gsem[b])

    return carry

  lax.fori_loop(0, GCH // NB_G, body, 0)
  for b in range(NB_G):
    pltpu.make_async_copy(
        rows[b], out_hbm.at[pl.ds((w * GCH + GCH - NB_G + b) * GK, GK)],
        osem[b]).wait()


def _tc1_body(parts_ref, emb_ref, u_ref, dinv_ref):
  ones = jnp.ones((NW, 1), jnp.float32)
  deg = lax.dot_general(
      parts_ref[...], ones, (((0,), (0,)), ((), ())),
      preferred_element_type=jnp.float32) + 1.0
  dinv = lax.rsqrt(deg)
  u_ref[...] = emb_ref[...] * dinv
  dinv_ref[...] = dinv


def _tc1(parts, emb):
  return pl.pallas_call(
      _tc1_body,
      out_shape=[
          jax.ShapeDtypeStruct((N, D), jnp.float32),
          jax.ShapeDtypeStruct((N, 1), jnp.float32),
      ],
  )(parts, emb)


def _tcc_body(parts_ref, c_ref):
  ones = jnp.ones((NW, 1), jnp.float32)
  c_ref[...] = lax.dot_general(
      parts_ref[...], ones, (((0,), (0,)), ((), ())),
      preferred_element_type=jnp.float32)


def _tcc(parts):
  return pl.pallas_call(
      _tcc_body,
      out_shape=jax.ShapeDtypeStruct((N, 1), jnp.float32),
  )(parts)


def _tc2_body(acc_ref, u_ref, dinv_ref, c_ref, w1_ref, b1_ref, ttab_ref, r_ref):
  s = acc_ref[0] + acc_ref[1]
  dinv = dinv_ref[...]
  su = s + u_ref[...]
  b1 = b1_ref[...]
  h = dinv * jnp.dot(su, w1_ref[...], preferred_element_type=jnp.float32) + b1
  ttab_ref[...] = (dinv * dinv) * su
  r_ref[...] = dinv * h + c_ref[...] * b1


def _tc2(accB, u, dinvcol, ccol, W1, b1row):
  nb = N // ROWBLK
  return pl.pallas_call(
      _tc2_body,
      grid=(nb,),
      in_specs=[
          pl.BlockSpec((NC, ROWBLK, D), lambda i: (0, i, 0)),
          pl.BlockSpec((ROWBLK, D), lambda i: (i, 0)),
          pl.BlockSpec((ROWBLK, 1), lambda i: (i, 0)),
          pl.BlockSpec((ROWBLK, 1), lambda i: (i, 0)),
          pl.BlockSpec((D, 2 * D), lambda i: (0, 0)),
          pl.BlockSpec((1, 2 * D), lambda i: (0, 0)),
      ],
      out_specs=[
          pl.BlockSpec((ROWBLK, D), lambda i: (i, 0)),
          pl.BlockSpec((ROWBLK, 2 * D), lambda i: (i, 0)),
      ],
      out_shape=[
          jax.ShapeDtypeStruct((N, D), jnp.float32),
          jax.ShapeDtypeStruct((N, 2 * D), jnp.float32),
      ],
  )(accB, u, dinvcol, ccol, W1, b1row)


def _tc3_body(acc_ref, r_ref, dinv_ref, w1_ref, w2_ref, b2_ref, g_ref):
  s2a = acc_ref[0] + acc_ref[1]
  z = jnp.dot(s2a, w1_ref[...], preferred_element_type=jnp.float32) + r_ref[...]
  g_ref[...] = dinv_ref[...] * jnp.dot(
      z, w2_ref[...], preferred_element_type=jnp.float32) + b2_ref[...]


def _tc3(accC, r, dinvcol, W1, W2, b2row):
  nb = N // ROWBLK
  return pl.pallas_call(
      _tc3_body,
      grid=(nb,),
      in_specs=[
          pl.BlockSpec((NC, ROWBLK, D), lambda i: (0, i, 0)),
          pl.BlockSpec((ROWBLK, 2 * D), lambda i: (i, 0)),
          pl.BlockSpec((ROWBLK, 1), lambda i: (i, 0)),
          pl.BlockSpec((D, 2 * D), lambda i: (0, 0)),
          pl.BlockSpec((2 * D, D), lambda i: (0, 0)),
          pl.BlockSpec((1, D), lambda i: (0, 0)),
      ],
      out_specs=pl.BlockSpec((ROWBLK, D), lambda i: (i, 0)),
      out_shape=jax.ShapeDtypeStruct((N, D), jnp.float32),
  )(accC, r, dinvcol, W1, W2, b2row)


def kernel(input, input_timestamp, input_id, edge_index, emb, W1, b1, W2, b2):
  del input_timestamp, input_id
  src = edge_index[0].astype(jnp.int32)
  dst = edge_index[1].astype(jnp.int32)
  src2 = src.reshape(E // K, K)
  dst2 = dst.reshape(E // K, K)

  deg_parts = _deg_pass(dst)
  u, dinvcol = _tc1(deg_parts, emb)
  accB = _segsum(u, src2, dst2)
  c_parts = _c_pass(dinvcol.reshape(N), src, dst)
  ccol = _tcc(c_parts)
  ttab, r = _tc2(accB, u, dinvcol, ccol, W1, b1.reshape(1, -1))
  accC = _segsum(ttab, src2, dst2)
  g = _tc3(accC, r, dinvcol, W1, W2, b2.reshape(1, -1))

  bsz, mlen = input.shape
  idx2 = input.reshape(-1).astype(jnp.int32).reshape(-1, GK)
  out = _gather_pass(g, idx2)
  return out.reshape(bsz, mlen, D)


# speedup vs baseline: 1.2410x; 1.1179x over previous
import functools

import jax
import jax.numpy as jnp
from jax import lax
from jax.experimental import pallas as pl
from jax.experimental.pallas import tpu as pltpu
from jax.experimental.pallas import tpu_sc as plsc

N = 10000
D = 128
E = 320000
NC, NS = 2, 16
NW = NC * NS
EW = E // NW
K = 50
NCH = EW // K
RPT = N // NS
VL = 16
NBR = 5
NFULL = (NCH // NBR) * NBR
REM = NCH - NFULL
NB_G = 4

GK = 100
GCH = (1024 * 50) // (NW * GK)
ROWBLK = 1000

_MESH = dict(core_axis_name="c", subcore_axis_name="s")
_SC_PARAMS = pltpu.CompilerParams(
    needs_layout_passes=False, use_tc_tiling_on_sc=False)


def _worker_id():
  return lax.axis_index("c") * NS + lax.axis_index("s")


@functools.partial(
    pl.kernel,
    out_type=jax.ShapeDtypeStruct((NW, N), jnp.float32),
    mesh=plsc.VectorSubcoreMesh(**_MESH),
    compiler_params=_SC_PARAMS,
    scratch_types=[
        pltpu.VMEM((EW,), jnp.int32),
        pltpu.VMEM((N,), jnp.float32),
        pltpu.SemaphoreType.DMA,
    ],
)
def _deg_pass(dst_hbm, out_hbm, idx_v, deg_v, sem):
  w = _worker_id()
  cp = pltpu.async_copy(dst_hbm.at[pl.ds(w * EW, EW)], idx_v, sem)

  def zero(i, carry):
    deg_v[pl.ds(i * VL, VL)] = jnp.zeros((VL,), jnp.float32)
    return carry

  lax.fori_loop(0, N // VL, zero, 0)
  cp.wait()

  ones = jnp.ones((VL,), jnp.float32)

  def scat(i, carry):
    idx = idx_v[pl.ds(i * VL, VL)]
    plsc.addupdate_scatter(deg_v, [idx], ones)
    return carry

  lax.fori_loop(0, EW // VL, scat, 0)
  pltpu.sync_copy(deg_v, out_hbm.at[w])


@functools.partial(
    pl.kernel,
    out_type=jax.ShapeDtypeStruct((NW, N), jnp.float32),
    mesh=plsc.VectorSubcoreMesh(**_MESH),
    compiler_params=_SC_PARAMS,
    scratch_types=[
        pltpu.VMEM((EW,), jnp.int32),
        pltpu.VMEM((EW,), jnp.int32),
        pltpu.VMEM((N,), jnp.float32),
        pltpu.VMEM((N,), jnp.float32),
        pltpu.SemaphoreType.DMA,
    ],
)
def _c_pass(dinv_hbm, src_hbm, dst_hbm, out_hbm, sidx, didx, dv, cacc, sem):
  w = _worker_id()
  cp0 = pltpu.async_copy(dinv_hbm, dv, sem)
  cp1 = pltpu.async_copy(src_hbm.at[pl.ds(w * EW, EW)], sidx, sem)
  cp2 = pltpu.async_copy(dst_hbm.at[pl.ds(w * EW, EW)], didx, sem)

  def zero(i, carry):
    cacc[pl.ds(i * VL, VL)] = jnp.zeros((VL,), jnp.float32)
    return carry

  lax.fori_loop(0, N // VL, zero, 0)
  cp0.wait()
  cp1.wait()
  cp2.wait()

  def scat(i, carry):
    si = sidx[pl.ds(i * VL, VL)]
    vals = plsc.load_gather(dv, [si])
    di = didx[pl.ds(i * VL, VL)]
    plsc.addupdate_scatter(cacc, [di], vals)
    return carry

  lax.fori_loop(0, EW // VL, scat, 0)
  pltpu.sync_copy(cacc, out_hbm.at[w])


@functools.partial(
    pl.kernel,
    out_type=jax.ShapeDtypeStruct((NC, N, D), jnp.float32),
    mesh=plsc.VectorSubcoreMesh(**_MESH),
    compiler_params=_SC_PARAMS,
    scratch_types=(
        [pltpu.VMEM_SHARED((N, D), jnp.float32),
         pltpu.VMEM((NCH, K), jnp.int32)]
        + [pltpu.VMEM((K,), jnp.int32) for _ in range(NBR)]
        + [pltpu.VMEM((K, D), jnp.float32) for _ in range(NBR)]
        + [pltpu.SemaphoreType.DMA for _ in range(3 * NBR + 2)]
    ),
)
def _segsum(tab_hbm, src2_hbm, dst2_hbm, out_hbm, acc, sidx, *rs):
  dbuf = rs[:NBR]
  rows = rs[NBR:2 * NBR]
  xsem = rs[2 * NBR:3 * NBR]
  gsem = rs[3 * NBR:4 * NBR]
  ssem = rs[4 * NBR:5 * NBR]
  isem = rs[5 * NBR]
  zsem = rs[5 * NBR + 1]
  c = lax.axis_index("c")
  s = lax.axis_index("s")
  w = c * NS + s
  ch0 = w * NCH
  cp_s = pltpu.async_copy(src2_hbm.at[pl.ds(ch0, NCH)], sidx, isem)

  def dbuf_start(j, b):
    pltpu.async_copy(dst2_hbm.at[ch0 + j], dbuf[b], xsem[b])

  def dbuf_wait(j, b):
    pltpu.make_async_copy(dst2_hbm.at[ch0 + j], dbuf[b], xsem[b]).wait()

  def gather_start(j, b):
    pltpu.async_copy(tab_hbm.at[sidx.at[j]], rows[b], gsem[b])

  def gather_wait(j, b):
    pltpu.make_async_copy(tab_hbm.at[sidx.at[j]], rows[b], gsem[b]).wait()

  def scat_start(b):
    pltpu.async_copy(rows[b], acc.at[dbuf[b]], ssem[b], add=True)

  def scat_wait(b):
    pltpu.make_async_copy(rows[b], acc.at[dbuf[b]], ssem[b]).wait()

  for b in range(NBR):
    dbuf_start(b, b)

  def zrow(i, carry):
    for ch in range(D // VL):
      rows[0][i, pl.ds(ch * VL, VL)] = jnp.zeros((VL,), jnp.float32)
    return carry

  lax.fori_loop(0, K, zrow, 0)
  row0 = s * RPT
  zc = []
  for r in range(RPT // K):
    zc.append(pltpu.async_copy(
        rows[0], acc.at[pl.ds(row0 + r * K, K)], zsem))
  if RPT % K:
    zc.append(pltpu.async_copy(
        rows[0].at[pl.ds(0, RPT % K)],
        acc.at[pl.ds(row0 + (RPT // K) * K, RPT % K)], zsem))
  cp_s.wait()
  for b in range(1, NBR):
    gather_start(b, b)
  for z in zc:
    z.wait()
  gather_start(0, 0)
  plsc.subcore_barrier()

  def body(jj, carry):
    base = jj * NBR
    for b in range(NBR):
      gather_wait(base + b, b)
      dbuf_wait(base + b, b)
      scat_start(b)
    for b in range(NBR):
      nxt = base + NBR + b

      @pl.when(nxt < NCH)
      def _():
        scat_wait(b)
        dbuf_start(nxt, b)
        gather_start(nxt, b)

    return carry

  lax.fori_loop(0, NCH // NBR, body, 0)
  for t in range(REM):
    gather_wait(NFULL + t, t)
    dbuf_wait(NFULL + t, t)
    scat_start(t)
  for b in range(NBR):
    scat_wait(b)
  plsc.subcore_barrier()
  pltpu.sync_copy(acc.at[pl.ds(row0, RPT)], out_hbm.at[c, pl.ds(row0, RPT)])


@functools.partial(
    pl.kernel,
    out_type=jax.ShapeDtypeStruct((1024 * 50, D), jnp.float32),
    mesh=plsc.VectorSubcoreMesh(**_MESH),
    compiler_params=_SC_PARAMS,
    scratch_types=(
        [pltpu.VMEM((GCH, GK), jnp.int32)]
        + [pltpu.VMEM((GK, D), jnp.float32) for _ in range(NB_G)]
        + [pltpu.SemaphoreType.DMA for _ in range(2 * NB_G)]
    ),
)
def _gather_pass(g_hbm, idx2_hbm, out_hbm, idxv, *rs):
  rows = rs[:NB_G]
  gsem = rs[NB_G:2 * NB_G]
  osem = rs[2 * NB_G:3 * NB_G]
  w = _worker_id()
  pltpu.sync_copy(idx2_hbm.at[pl.ds(w * GCH, GCH)], idxv)
  for b in range(NB_G):
    pltpu.async_copy(g_hbm.at[idxv.at[b]], rows[b], gsem[b])

  def body(jj, carry):
    base = jj * NB_G
    for b in range(NB_G):
      pltpu.make_async_copy(g_hbm.at[idxv.at[base + b]], rows[b],
                            gsem[b]).wait()
      pltpu.async_copy(
          rows[b], out_hbm.at[pl.ds((w * GCH + base + b) * GK, GK)], osem[b])
    for b in range(NB_G):
      nxt = base + NB_G + b

      @pl.when(nxt < GCH)
      def _():
        pltpu.make_async_copy(
            rows[b], out_hbm.at[pl.ds((w * GCH + base + b) * GK, GK)],
            osem[b]).wait()
        pltpu.async_copy(g_hbm.at[idxv.at[nxt]], rows[b], gsem[b])

    return carry

  lax.fori_loop(0, GCH // NB_G, body, 0)
  for b in range(NB_G):
    pltpu.make_async_copy(
        rows[b], out_hbm.at[pl.ds((w * GCH + GCH - NB_G + b) * GK, GK)],
        osem[b]).wait()


def _tc1_body(parts_ref, emb_ref, u_ref, dinv_ref):
  ones = jnp.ones((NW, 1), jnp.float32)
  deg = lax.dot_general(
      parts_ref[...], ones, (((0,), (0,)), ((), ())),
      preferred_element_type=jnp.float32) + 1.0
  dinv = lax.rsqrt(deg)
  u_ref[...] = emb_ref[...] * dinv
  dinv_ref[...] = dinv


def _tc1(parts, emb):
  return pl.pallas_call(
      _tc1_body,
      out_shape=[
          jax.ShapeDtypeStruct((N, D), jnp.float32),
          jax.ShapeDtypeStruct((N, 1), jnp.float32),
      ],
  )(parts, emb)


def _tcc_body(parts_ref, c_ref):
  ones = jnp.ones((NW, 1), jnp.float32)
  c_ref[...] = lax.dot_general(
      parts_ref[...], ones, (((0,), (0,)), ((), ())),
      preferred_element_type=jnp.float32)


def _tcc(parts):
  return pl.pallas_call(
      _tcc_body,
      out_shape=jax.ShapeDtypeStruct((N, 1), jnp.float32),
  )(parts)


def _tc2_body(acc_ref, u_ref, dinv_ref, c_ref, w1_ref, b1_ref, ttab_ref, r_ref):
  s = acc_ref[0] + acc_ref[1]
  dinv = dinv_ref[...]
  su = s + u_ref[...]
  b1 = b1_ref[...]
  h = dinv * jnp.dot(su, w1_ref[...], preferred_element_type=jnp.float32) + b1
  ttab_ref[...] = (dinv * dinv) * su
  r_ref[...] = dinv * h + c_ref[...] * b1


def _tc2(accB, u, dinvcol, ccol, W1, b1row):
  nb = N // ROWBLK
  return pl.pallas_call(
      _tc2_body,
      grid=(nb,),
      in_specs=[
          pl.BlockSpec((NC, ROWBLK, D), lambda i: (0, i, 0)),
          pl.BlockSpec((ROWBLK, D), lambda i: (i, 0)),
          pl.BlockSpec((ROWBLK, 1), lambda i: (i, 0)),
          pl.BlockSpec((ROWBLK, 1), lambda i: (i, 0)),
          pl.BlockSpec((D, 2 * D), lambda i: (0, 0)),
          pl.BlockSpec((1, 2 * D), lambda i: (0, 0)),
      ],
      out_specs=[
          pl.BlockSpec((ROWBLK, D), lambda i: (i, 0)),
          pl.BlockSpec((ROWBLK, 2 * D), lambda i: (i, 0)),
      ],
      out_shape=[
          jax.ShapeDtypeStruct((N, D), jnp.float32),
          jax.ShapeDtypeStruct((N, 2 * D), jnp.float32),
      ],
  )(accB, u, dinvcol, ccol, W1, b1row)


def _tc3_body(acc_ref, r_ref, dinv_ref, w1_ref, w2_ref, b2_ref, g_ref):
  s2a = acc_ref[0] + acc_ref[1]
  z = jnp.dot(s2a, w1_ref[...], preferred_element_type=jnp.float32) + r_ref[...]
  g_ref[...] = dinv_ref[...] * jnp.dot(
      z, w2_ref[...], preferred_element_type=jnp.float32) + b2_ref[...]


def _tc3(accC, r, dinvcol, W1, W2, b2row):
  nb = N // ROWBLK
  return pl.pallas_call(
      _tc3_body,
      grid=(nb,),
      in_specs=[
          pl.BlockSpec((NC, ROWBLK, D), lambda i: (0, i, 0)),
          pl.BlockSpec((ROWBLK, 2 * D), lambda i: (i, 0)),
          pl.BlockSpec((ROWBLK, 1), lambda i: (i, 0)),
          pl.BlockSpec((D, 2 * D), lambda i: (0, 0)),
          pl.BlockSpec((2 * D, D), lambda i: (0, 0)),
          pl.BlockSpec((1, D), lambda i: (0, 0)),
      ],
      out_specs=pl.BlockSpec((ROWBLK, D), lambda i: (i, 0)),
      out_shape=jax.ShapeDtypeStruct((N, D), jnp.float32),
  )(accC, r, dinvcol, W1, W2, b2row)


def kernel(input, input_timestamp, input_id, edge_index, emb, W1, b1, W2, b2):
  del input_timestamp, input_id
  src = edge_index[0].astype(jnp.int32)
  dst = edge_index[1].astype(jnp.int32)
  src2 = src.reshape(E // K, K)
  dst2 = dst.reshape(E // K, K)

  deg_parts = _deg_pass(dst)
  u, dinvcol = _tc1(deg_parts, emb)
  accB = _segsum(u, src2, dst2)
  c_parts = _c_pass(dinvcol.reshape(N), src, dst)
  ccol = _tcc(c_parts)
  ttab, r = _tc2(accB, u, dinvcol, ccol, W1, b1.reshape(1, -1))
  accC = _segsum(ttab, src2, dst2)
  g = _tc3(accC, r, dinvcol, W1, W2, b2.reshape(1, -1))

  bsz, mlen = input.shape
  idx2 = input.reshape(-1).astype(jnp.int32).reshape(-1, GK)
  out = _gather_pass(g, idx2)
  return out.reshape(bsz, mlen, D)


# speedup vs baseline: 1.2593x; 1.0148x over previous
import functools

import jax
import jax.numpy as jnp
from jax import lax
from jax.experimental import pallas as pl
from jax.experimental.pallas import tpu as pltpu
from jax.experimental.pallas import tpu_sc as plsc

N = 10000
D = 128
E = 320000
NC, NS = 2, 16
NW = NC * NS
EW = E // NW
K = 50
NCH = EW // K
RPT = N // NS
VL = 16
NBR = 6
NFULL = (NCH // NBR) * NBR
REM = NCH - NFULL
NB_G = 4

GK = 100
GCH = (1024 * 50) // (NW * GK)
ROWBLK = 1000

_MESH = dict(core_axis_name="c", subcore_axis_name="s")
_SC_PARAMS = pltpu.CompilerParams(
    needs_layout_passes=False, use_tc_tiling_on_sc=False)


def _worker_id():
  return lax.axis_index("c") * NS + lax.axis_index("s")


@functools.partial(
    pl.kernel,
    out_type=jax.ShapeDtypeStruct((NW, N), jnp.float32),
    mesh=plsc.VectorSubcoreMesh(**_MESH),
    compiler_params=_SC_PARAMS,
    scratch_types=[
        pltpu.VMEM((EW,), jnp.int32),
        pltpu.VMEM((N,), jnp.float32),
        pltpu.SemaphoreType.DMA,
    ],
)
def _deg_pass(dst_hbm, out_hbm, idx_v, deg_v, sem):
  w = _worker_id()
  cp = pltpu.async_copy(dst_hbm.at[pl.ds(w * EW, EW)], idx_v, sem)

  def zero(i, carry):
    deg_v[pl.ds(i * VL, VL)] = jnp.zeros((VL,), jnp.float32)
    return carry

  lax.fori_loop(0, N // VL, zero, 0)
  cp.wait()

  ones = jnp.ones((VL,), jnp.float32)

  def scat(i, carry):
    idx = idx_v[pl.ds(i * VL, VL)]
    plsc.addupdate_scatter(deg_v, [idx], ones)
    return carry

  lax.fori_loop(0, EW // VL, scat, 0)
  pltpu.sync_copy(deg_v, out_hbm.at[w])


@functools.partial(
    pl.kernel,
    out_type=jax.ShapeDtypeStruct((NW, N), jnp.float32),
    mesh=plsc.VectorSubcoreMesh(**_MESH),
    compiler_params=_SC_PARAMS,
    scratch_types=[
        pltpu.VMEM((EW,), jnp.int32),
        pltpu.VMEM((EW,), jnp.int32),
        pltpu.VMEM((N,), jnp.float32),
        pltpu.VMEM((N,), jnp.float32),
        pltpu.SemaphoreType.DMA,
    ],
)
def _c_pass(dinv_hbm, src_hbm, dst_hbm, out_hbm, sidx, didx, dv, cacc, sem):
  w = _worker_id()
  cp0 = pltpu.async_copy(dinv_hbm, dv, sem)
  cp1 = pltpu.async_copy(src_hbm.at[pl.ds(w * EW, EW)], sidx, sem)
  cp2 = pltpu.async_copy(dst_hbm.at[pl.ds(w * EW, EW)], didx, sem)

  def zero(i, carry):
    cacc[pl.ds(i * VL, VL)] = jnp.zeros((VL,), jnp.float32)
    return carry

  lax.fori_loop(0, N // VL, zero, 0)
  cp0.wait()
  cp1.wait()
  cp2.wait()

  def scat(i, carry):
    si = sidx[pl.ds(i * VL, VL)]
    vals = plsc.load_gather(dv, [si])
    di = didx[pl.ds(i * VL, VL)]
    plsc.addupdate_scatter(cacc, [di], vals)
    return carry

  lax.fori_loop(0, EW // VL, scat, 0)
  pltpu.sync_copy(cacc, out_hbm.at[w])


@functools.partial(
    pl.kernel,
    out_type=jax.ShapeDtypeStruct((NC, N, D), jnp.float32),
    mesh=plsc.VectorSubcoreMesh(**_MESH),
    compiler_params=_SC_PARAMS,
    scratch_types=(
        [pltpu.VMEM_SHARED((N, D), jnp.float32),
         pltpu.VMEM((NCH, K), jnp.int32)]
        + [pltpu.VMEM((K,), jnp.int32) for _ in range(NBR)]
        + [pltpu.VMEM((K, D), jnp.float32) for _ in range(NBR)]
        + [pltpu.SemaphoreType.DMA for _ in range(3 * NBR + 2)]
    ),
)
def _segsum(tab_hbm, src2_hbm, dst2_hbm, out_hbm, acc, sidx, *rs):
  dbuf = rs[:NBR]
  rows = rs[NBR:2 * NBR]
  xsem = rs[2 * NBR:3 * NBR]
  gsem = rs[3 * NBR:4 * NBR]
  ssem = rs[4 * NBR:5 * NBR]
  isem = rs[5 * NBR]
  zsem = rs[5 * NBR + 1]
  c = lax.axis_index("c")
  s = lax.axis_index("s")
  w = c * NS + s
  ch0 = w * NCH
  cp_s = pltpu.async_copy(src2_hbm.at[pl.ds(ch0, NCH)], sidx, isem)

  def dbuf_start(j, b):
    pltpu.async_copy(dst2_hbm.at[ch0 + j], dbuf[b], xsem[b])

  def dbuf_wait(j, b):
    pltpu.make_async_copy(dst2_hbm.at[ch0 + j], dbuf[b], xsem[b]).wait()

  def gather_start(j, b):
    pltpu.async_copy(tab_hbm.at[sidx.at[j]], rows[b], gsem[b])

  def gather_wait(j, b):
    pltpu.make_async_copy(tab_hbm.at[sidx.at[j]], rows[b], gsem[b]).wait()

  def scat_start(b):
    pltpu.async_copy(rows[b], acc.at[dbuf[b]], ssem[b], add=True)

  def scat_wait(b):
    pltpu.make_async_copy(rows[b], acc.at[dbuf[b]], ssem[b]).wait()

  for b in range(NBR):
    dbuf_start(b, b)

  def zrow(i, carry):
    for ch in range(D // VL):
      rows[0][i, pl.ds(ch * VL, VL)] = jnp.zeros((VL,), jnp.float32)
    return carry

  lax.fori_loop(0, K, zrow, 0)
  row0 = s * RPT
  zc = []
  for r in range(RPT // K):
    zc.append(pltpu.async_copy(
        rows[0], acc.at[pl.ds(row0 + r * K, K)], zsem))
  if RPT % K:
    zc.append(pltpu.async_copy(
        rows[0].at[pl.ds(0, RPT % K)],
        acc.at[pl.ds(row0 + (RPT // K) * K, RPT % K)], zsem))
  cp_s.wait()
  for b in range(1, NBR):
    gather_start(b, b)
  for z in zc:
    z.wait()
  gather_start(0, 0)
  plsc.subcore_barrier()

  def body(jj, carry):
    base = jj * NBR
    for b in range(NBR):
      gather_wait(base + b, b)
      dbuf_wait(base + b, b)
      scat_start(b)
    for b in range(NBR):
      nxt = base + NBR + b

      @pl.when(nxt < NCH)
      def _():
        scat_wait(b)
        dbuf_start(nxt, b)
        gather_start(nxt, b)

    return carry

  lax.fori_loop(0, NCH // NBR, body, 0)
  for t in range(REM):
    gather_wait(NFULL + t, t)
    dbuf_wait(NFULL + t, t)
    scat_start(t)
  for b in range(NBR):
    scat_wait(b)
  plsc.subcore_barrier()
  pltpu.sync_copy(acc.at[pl.ds(row0, RPT)], out_hbm.at[c, pl.ds(row0, RPT)])


@functools.partial(
    pl.kernel,
    out_type=jax.ShapeDtypeStruct((1024 * 50, D), jnp.float32),
    mesh=plsc.VectorSubcoreMesh(**_MESH),
    compiler_params=_SC_PARAMS,
    scratch_types=(
        [pltpu.VMEM((GCH, GK), jnp.int32)]
        + [pltpu.VMEM((GK, D), jnp.float32) for _ in range(NB_G)]
        + [pltpu.SemaphoreType.DMA for _ in range(2 * NB_G)]
    ),
)
def _gather_pass(g_hbm, idx2_hbm, out_hbm, idxv, *rs):
  rows = rs[:NB_G]
  gsem = rs[NB_G:2 * NB_G]
  osem = rs[2 * NB_G:3 * NB_G]
  w = _worker_id()
  pltpu.sync_copy(idx2_hbm.at[pl.ds(w * GCH, GCH)], idxv)
  for b in range(NB_G):
    pltpu.async_copy(g_hbm.at[idxv.at[b]], rows[b], gsem[b])

  def body(jj, carry):
    base = jj * NB_G
    for b in range(NB_G):
      pltpu.make_async_copy(g_hbm.at[idxv.at[base + b]], rows[b],
                            gsem[b]).wait()
      pltpu.async_copy(
          rows[b], out_hbm.at[pl.ds((w * GCH + base + b) * GK, GK)], osem[b])
    for b in range(NB_G):
      nxt = base + NB_G + b

      @pl.when(nxt < GCH)
      def _():
        pltpu.make_async_copy(
            rows[b], out_hbm.at[pl.ds((w * GCH + base + b) * GK, GK)],
            osem[b]).wait()
        pltpu.async_copy(g_hbm.at[idxv.at[nxt]], rows[b], gsem[b])

    return carry

  lax.fori_loop(0, GCH // NB_G, body, 0)
  for b in range(NB_G):
    pltpu.make_async_copy(
        rows[b], out_hbm.at[pl.ds((w * GCH + GCH - NB_G + b) * GK, GK)],
        osem[b]).wait()


def _tc1_body(parts_ref, emb_ref, u_ref, dinv_ref):
  ones = jnp.ones((NW, 1), jnp.float32)
  deg = lax.dot_general(
      parts_ref[...], ones, (((0,), (0,)), ((), ())),
      preferred_element_type=jnp.float32) + 1.0
  dinv = lax.rsqrt(deg)
  u_ref[...] = emb_ref[...] * dinv
  dinv_ref[...] = dinv


def _tc1(parts, emb):
  return pl.pallas_call(
      _tc1_body,
      out_shape=[
          jax.ShapeDtypeStruct((N, D), jnp.float32),
          jax.ShapeDtypeStruct((N, 1), jnp.float32),
      ],
  )(parts, emb)


def _tcc_body(parts_ref, c_ref):
  ones = jnp.ones((NW, 1), jnp.float32)
  c_ref[...] = lax.dot_general(
      parts_ref[...], ones, (((0,), (0,)), ((), ())),
      preferred_element_type=jnp.float32)


def _tcc(parts):
  return pl.pallas_call(
      _tcc_body,
      out_shape=jax.ShapeDtypeStruct((N, 1), jnp.float32),
  )(parts)


def _tc2_body(acc_ref, u_ref, dinv_ref, c_ref, w1_ref, b1_ref, ttab_ref, r_ref):
  s = acc_ref[0] + acc_ref[1]
  dinv = dinv_ref[...]
  su = s + u_ref[...]
  b1 = b1_ref[...]
  h = dinv * jnp.dot(su, w1_ref[...], preferred_element_type=jnp.float32) + b1
  ttab_ref[...] = (dinv * dinv) * su
  r_ref[...] = dinv * h + c_ref[...] * b1


def _tc2(accB, u, dinvcol, ccol, W1, b1row):
  nb = N // ROWBLK
  return pl.pallas_call(
      _tc2_body,
      grid=(nb,),
      in_specs=[
          pl.BlockSpec((NC, ROWBLK, D), lambda i: (0, i, 0)),
          pl.BlockSpec((ROWBLK, D), lambda i: (i, 0)),
          pl.BlockSpec((ROWBLK, 1), lambda i: (i, 0)),
          pl.BlockSpec((ROWBLK, 1), lambda i: (i, 0)),
          pl.BlockSpec((D, 2 * D), lambda i: (0, 0)),
          pl.BlockSpec((1, 2 * D), lambda i: (0, 0)),
      ],
      out_specs=[
          pl.BlockSpec((ROWBLK, D), lambda i: (i, 0)),
          pl.BlockSpec((ROWBLK, 2 * D), lambda i: (i, 0)),
      ],
      out_shape=[
          jax.ShapeDtypeStruct((N, D), jnp.float32),
          jax.ShapeDtypeStruct((N, 2 * D), jnp.float32),
      ],
  )(accB, u, dinvcol, ccol, W1, b1row)


def _tc3_body(acc_ref, r_ref, dinv_ref, w1_ref, w2_ref, b2_ref, g_ref):
  s2a = acc_ref[0] + acc_ref[1]
  z = jnp.dot(s2a, w1_ref[...], preferred_element_type=jnp.float32) + r_ref[...]
  g_ref[...] = dinv_ref[...] * jnp.dot(
      z, w2_ref[...], preferred_element_type=jnp.float32) + b2_ref[...]


def _tc3(accC, r, dinvcol, W1, W2, b2row):
  nb = N // ROWBLK
  return pl.pallas_call(
      _tc3_body,
      grid=(nb,),
      in_specs=[
          pl.BlockSpec((NC, ROWBLK, D), lambda i: (0, i, 0)),
          pl.BlockSpec((ROWBLK, 2 * D), lambda i: (i, 0)),
          pl.BlockSpec((ROWBLK, 1), lambda i: (i, 0)),
          pl.BlockSpec((D, 2 * D), lambda i: (0, 0)),
          pl.BlockSpec((2 * D, D), lambda i: (0, 0)),
          pl.BlockSpec((1, D), lambda i: (0, 0)),
      ],
      out_specs=pl.BlockSpec((ROWBLK, D), lambda i: (i, 0)),
      out_shape=jax.ShapeDtypeStruct((N, D), jnp.float32),
  )(accC, r, dinvcol, W1, W2, b2row)


def kernel(input, input_timestamp, input_id, edge_index, emb, W1, b1, W2, b2):
  del input_timestamp, input_id
  src = edge_index[0].astype(jnp.int32)
  dst = edge_index[1].astype(jnp.int32)
  src2 = src.reshape(E // K, K)
  dst2 = dst.reshape(E // K, K)

  deg_parts = _deg_pass(dst)
  u, dinvcol = _tc1(deg_parts, emb)
  accB = _segsum(u, src2, dst2)
  c_parts = _c_pass(dinvcol.reshape(N), src, dst)
  ccol = _tcc(c_parts)
  ttab, r = _tc2(accB, u, dinvcol, ccol, W1, b1.reshape(1, -1))
  accC = _segsum(ttab, src2, dst2)
  g = _tc3(accC, r, dinvcol, W1, W2, b2.reshape(1, -1))

  bsz, mlen = input.shape
  idx2 = input.reshape(-1).astype(jnp.int32).reshape(-1, GK)
  out = _gather_pass(g, idx2)
  return out.reshape(bsz, mlen, D)
